# baseline (speedup 1.0000x reference)
import functools

import jax
import jax.numpy as jnp
from jax import lax
from jax.experimental import pallas as pl
from jax.experimental.pallas import tpu as pltpu
from jax.experimental.pallas import tpu_sc as plsc

_K = 6
_A = 4


def _gather_kernel(n_workers, n_chunks, chunk, d):
  mesh = plsc.VectorSubcoreMesh(core_axis_name="c", subcore_axis_name="s")
  n_loop = (n_chunks - 2 * _K) // _K
  tail_start = _K + _K * n_loop
  assert tail_start + _A <= n_chunks

  @functools.partial(
      pl.kernel,
      mesh=mesh,
      out_type=jax.ShapeDtypeStruct(
          (n_workers * n_chunks, chunk, d), jnp.float32),
      scratch_types=[
          pltpu.VMEM((n_chunks, chunk), jnp.int32),
          pltpu.VMEM((_K, chunk, d), jnp.float32),
      ] + [pltpu.SemaphoreType.DMA] * (2 * _K),
  )
  def k(idx_hbm, table_hbm, out_hbm, idx_v, rows_v, *sems):
    gsems = sems[:_K]
    ssems = sems[_K:]
    wid = lax.axis_index("s") * 2 + lax.axis_index("c")
    base = wid * n_chunks
    pltpu.sync_copy(idx_hbm.at[wid], idx_v)

    def gather(j, b):
      return pltpu.make_async_copy(table_hbm.at[idx_v.at[j]], rows_v.at[b],
                                   gsems[b])

    def scatter(j, b):
      return pltpu.make_async_copy(rows_v.at[b], out_hbm.at[base + j],
                                   ssems[b])

    def body(j, b, scatter_wait, gather_ahead):
      bb = (b + _A) % _K
      if scatter_wait:
        scatter(j - (_K - _A), bb).wait()
      if gather_ahead:
        gather(j + _A, bb).start()
      gather(j, b).wait()
      scatter(j, b).start()

    for j in range(_A):
      gather(j, j).start()
    for j in range(_K):
      body(j, j, scatter_wait=j >= _K - _A, gather_ahead=True)

    def loop_body(u, _):
      j0 = _K + _K * u
      for i in range(_K):
        body(j0 + i, i, scatter_wait=True, gather_ahead=True)
      return 0

    lax.fori_loop(0, n_loop, loop_body, 0)

    for j in range(tail_start, n_chunks):
      body(j, j % _K, scatter_wait=True, gather_ahead=j + _A < n_chunks)
    for j in range(n_chunks - (_K - _A), n_chunks):
      scatter(j, j % _K).wait()

  return k


def kernel(inputs, table):
  b, s = inputs.shape
  v, d = table.shape
  n = b * s
  n_workers = 32
  chunk = 128
  n_per_w = n // n_workers
  n_chunks = n_per_w // chunk
  idx = inputs.astype(jnp.int32).reshape(n_workers, n_chunks, chunk)
  out = _gather_kernel(n_workers, n_chunks, chunk, d)(idx, table)
  return out.reshape(b, s, d)


# speedup vs baseline: 1.7843x; 1.7843x over previous
import functools

import jax
import jax.numpy as jnp
from jax import lax
from jax.experimental import pallas as pl
from jax.experimental.pallas import tpu as pltpu
from jax.experimental.pallas import tpu_sc as plsc

_K = 8
_A = 6


def _gather_kernel(n_workers, bpw, seq, d):
  mesh = plsc.VectorSubcoreMesh(core_axis_name="c", subcore_axis_name="s")
  n_chunks = bpw
  n_loop = (n_chunks - 2 * _K) // _K
  tail_start = _K + _K * n_loop
  assert tail_start + _A <= n_chunks

  @functools.partial(
      pl.kernel,
      mesh=mesh,
      out_type=jax.ShapeDtypeStruct(
          (n_workers * bpw, seq, d), jnp.float32),
      scratch_types=[
          pltpu.VMEM((n_chunks, seq), jnp.int32),
          pltpu.VMEM((_K, seq, d), jnp.float32),
      ] + [pltpu.SemaphoreType.DMA] * (2 * _K),
  )
  def k(idx_hbm, table_hbm, out_hbm, idx_v, rows_v, *sems):
    gsems = sems[:_K]
    ssems = sems[_K:]
    wid = lax.axis_index("s") * 2 + lax.axis_index("c")
    base = wid * bpw
    pltpu.sync_copy(idx_hbm.at[wid], idx_v)

    def gather(j, b):
      return pltpu.make_async_copy(table_hbm.at[idx_v.at[j]], rows_v.at[b],
                                   gsems[b])

    def scatter(j, b):
      return pltpu.make_async_copy(
          rows_v.at[b], out_hbm.at[base + j], ssems[b])

    def body(j, b, scatter_wait, gather_ahead):
      bb = (b + _A) % _K
      if scatter_wait:
        scatter(j - (_K - _A), bb).wait()
      if gather_ahead:
        gather(j + _A, bb).start()
      gather(j, b).wait()
      scatter(j, b).start()

    for j in range(_A):
      gather(j, j).start()
    for j in range(_K):
      body(j, j, scatter_wait=j >= _K - _A, gather_ahead=True)

    def loop_body(u, _):
      j0 = _K + _K * u
      for i in range(_K):
        body(j0 + i, i, scatter_wait=True, gather_ahead=True)
      return 0

    lax.fori_loop(0, n_loop, loop_body, 0)

    for j in range(tail_start, n_chunks):
      body(j, j % _K, scatter_wait=True, gather_ahead=j + _A < n_chunks)
    for j in range(n_chunks - (_K - _A), n_chunks):
      scatter(j, j % _K).wait()

  return k


def kernel(inputs, table):
  b, s = inputs.shape
  v, d = table.shape
  n_workers = 32
  bpw = b // n_workers
  idx = inputs.astype(jnp.int32).reshape(n_workers, bpw, s)
  return _gather_kernel(n_workers, bpw, s, d)(idx, table)


# speedup vs baseline: 3.2103x; 1.7991x over previous
import functools

import jax
import jax.numpy as jnp
from jax import lax
from jax.experimental import pallas as pl
from jax.experimental.pallas import tpu as pltpu
from jax.experimental.pallas import tpu_sc as plsc

_K = 6
_A = 4


def _gather_kernel(n_workers, seq, bpw, d):
  mesh = plsc.VectorSubcoreMesh(core_axis_name="c", subcore_axis_name="s")
  n_chunks = seq
  n_loop = (n_chunks - 2 * _K) // _K
  tail_start = _K + _K * n_loop
  assert tail_start + _A <= n_chunks

  @functools.partial(
      pl.kernel,
      mesh=mesh,
      out_type=jax.ShapeDtypeStruct(
          (seq, n_workers * bpw, d), jnp.float32),
      scratch_types=[
          pltpu.VMEM((n_chunks, bpw), jnp.int32),
          pltpu.VMEM((_K, bpw, d), jnp.float32),
      ] + [pltpu.SemaphoreType.DMA] * (2 * _K),
  )
  def k(idx_hbm, table_hbm, out_hbm, idx_v, rows_v, *sems):
    gsems = sems[:_K]
    ssems = sems[_K:]
    wid = lax.axis_index("s") * 2 + lax.axis_index("c")
    base = wid * bpw
    pltpu.sync_copy(idx_hbm.at[:, wid], idx_v)

    def gather(j, b):
      return pltpu.make_async_copy(table_hbm.at[idx_v.at[j]], rows_v.at[b],
                                   gsems[b])

    def scatter(j, b):
      return pltpu.make_async_copy(
          rows_v.at[b], out_hbm.at[j, pl.ds(base, bpw)], ssems[b])

    def body(j, b, scatter_wait, gather_ahead):
      bb = (b + _A) % _K
      if scatter_wait:
        scatter(j - (_K - _A), bb).wait()
      if gather_ahead:
        gather(j + _A, bb).start()
      gather(j, b).wait()
      scatter(j, b).start()

    for j in range(_A):
      gather(j, j).start()
    for j in range(_K):
      body(j, j, scatter_wait=j >= _K - _A, gather_ahead=True)

    def loop_body(u, _):
      j0 = _K + _K * u
      for i in range(_K):
        body(j0 + i, i, scatter_wait=True, gather_ahead=True)
      return 0

    lax.fori_loop(0, n_loop, loop_body, 0)

    for j in range(tail_start, n_chunks):
      body(j, j % _K, scatter_wait=True, gather_ahead=j + _A < n_chunks)
    for j in range(n_chunks - (_K - _A), n_chunks):
      scatter(j, j % _K).wait()

  return k


def kernel(inputs, table):
  b, s = inputs.shape
  v, d = table.shape
  n_workers = 32
  bpw = b // n_workers
  idx = jnp.transpose(inputs.astype(jnp.int32)).reshape(s, n_workers, bpw)
  out = _gather_kernel(n_workers, s, bpw, d)(idx, table)
  return jnp.swapaxes(out, 0, 1)


# speedup vs baseline: 3.2170x; 1.0021x over previous
import functools

import jax
import jax.numpy as jnp
from jax import lax
from jax.experimental import pallas as pl
from jax.experimental.pallas import tpu as pltpu
from jax.experimental.pallas import tpu_sc as plsc

_K = 7
_A = 5


def _gather_kernel(n_workers, seq, bpw, d):
  mesh = plsc.VectorSubcoreMesh(core_axis_name="c", subcore_axis_name="s")
  n_chunks = seq
  n_loop = (n_chunks - 2 * _K) // _K
  tail_start = _K + _K * n_loop
  assert tail_start + _A <= n_chunks

  @functools.partial(
      pl.kernel,
      mesh=mesh,
      out_type=jax.ShapeDtypeStruct(
          (seq, n_workers * bpw, d), jnp.float32),
      scratch_types=[
          pltpu.VMEM((n_chunks, bpw), jnp.int32),
          pltpu.VMEM((_K, bpw, d), jnp.float32),
      ] + [pltpu.SemaphoreType.DMA] * (2 * _K),
  )
  def k(idx_hbm, table_hbm, out_hbm, idx_v, rows_v, *sems):
    gsems = sems[:_K]
    ssems = sems[_K:]
    wid = lax.axis_index("s") * 2 + lax.axis_index("c")
    base = wid * bpw
    pltpu.sync_copy(idx_hbm.at[:, wid], idx_v)

    def gather(j, b):
      return pltpu.make_async_copy(table_hbm.at[idx_v.at[j]], rows_v.at[b],
                                   gsems[b])

    def scatter(j, b):
      return pltpu.make_async_copy(
          rows_v.at[b], out_hbm.at[j, pl.ds(base, bpw)], ssems[b])

    def body(j, b, scatter_wait, gather_ahead):
      bb = (b + _A) % _K
      if scatter_wait:
        scatter(j - (_K - _A), bb).wait()
      if gather_ahead:
        gather(j + _A, bb).start()
      gather(j, b).wait()
      scatter(j, b).start()

    for j in range(_A):
      gather(j, j).start()
    for j in range(_K):
      body(j, j, scatter_wait=j >= _K - _A, gather_ahead=True)

    def loop_body(u, _):
      j0 = _K + _K * u
      for i in range(_K):
        body(j0 + i, i, scatter_wait=True, gather_ahead=True)
      return 0

    lax.fori_loop(0, n_loop, loop_body, 0)

    for j in range(tail_start, n_chunks):
      body(j, j % _K, scatter_wait=True, gather_ahead=j + _A < n_chunks)
    for j in range(n_chunks - (_K - _A), n_chunks):
      scatter(j, j % _K).wait()

  return k


def kernel(inputs, table):
  b, s = inputs.shape
  v, d = table.shape
  n_workers = 32
  bpw = b // n_workers
  idx = jnp.transpose(inputs.astype(jnp.int32)).reshape(s, n_workers, bpw)
  out = _gather_kernel(n_workers, s, bpw, d)(idx, table)
  return jnp.swapaxes(out, 0, 1)
